# baseline (speedup 1.0000x reference)
import functools

import jax
import jax.numpy as jnp
from jax import lax
from jax.experimental import pallas as pl
from jax.experimental.pallas import tpu as pltpu
from jax.experimental.pallas import tpu_sc as plsc

_B = 4096
_L = 200
_D = 64
_HID = 256
_NCLASS = 4

_NC, _NS = 2, 16
_NW = _NC * _NS

_LP = 208
_CHUNK = _LP // 2
_ROWS = 2 * _B
_RPW = _ROWS // _NW
_NBUF = 2


def _sum_rows(buf, r, out_v, corr, scale):
    zeros = jnp.zeros((16,), jnp.float32)

    def body(j, acc):
        a = list(acc)
        for u in range(4):
            row = j * 4 + u
            for k in range(4):
                a[k] = a[k] + buf[row, pl.ds(k * 16, 16)]
        return tuple(a)

    acc = lax.fori_loop(0, _LP // 4, body, (zeros,) * 4)
    for k in range(4):
        out_v[r, pl.ds(k * 16, 16)] = (acc[k] - corr[k]) * scale


def _sc_avg_call(idx2, embed):
    mesh = plsc.VectorSubcoreMesh(core_axis_name="c", subcore_axis_name="s",
                                  num_cores=_NC, num_subcores=_NS)

    @functools.partial(
        pl.kernel,
        out_type=jax.ShapeDtypeStruct((_ROWS, _D), jnp.float32),
        mesh=mesh,
        compiler_params=pltpu.CompilerParams(use_tc_tiling_on_sc=False),
        scratch_types=[
            pltpu.VMEM((_RPW * 2, _CHUNK), jnp.int32),
            pltpu.VMEM((_NBUF, _LP, _D), jnp.float32),
            pltpu.VMEM((_RPW, _D), jnp.float32),
            pltpu.VMEM((1, _D), jnp.float32),
            pltpu.SemaphoreType.DMA,
            pltpu.SemaphoreType.DMA,
        ],
    )
    def sc_avg(idx_hbm, embed_hbm, out_hbm, idx_v, buf_v, out_v, z_v, s0, s1):
        sems = (s0, s1)
        wid = lax.axis_index("s") * _NC + lax.axis_index("c")
        chunk_base = wid * (_RPW * 2)
        row_base = wid * _RPW

        pltpu.sync_copy(idx_hbm.at[pl.ds(chunk_base, _RPW * 2)], idx_v)
        pltpu.sync_copy(embed_hbm.at[pl.ds(0, 1)], z_v)

        npad = jnp.float32(_LP - _L)
        corr = tuple(z_v[0, pl.ds(k * 16, 16)] * npad for k in range(4))
        scale = jnp.float32(1.0 / _L)

        def fire(r, b):
            for c in range(2):
                pltpu.async_copy(
                    embed_hbm.at[idx_v.at[2 * r + c]],
                    buf_v.at[b, pl.ds(c * _CHUNK, _CHUNK)],
                    sems[b],
                )

        def wait(r, b):
            for c in range(2):
                pltpu.make_async_copy(
                    embed_hbm.at[idx_v.at[2 * r + c]],
                    buf_v.at[b, pl.ds(c * _CHUNK, _CHUNK)],
                    sems[b],
                ).wait()

        for b in range(_NBUF - 1):
            fire(b, b)

        @pl.loop(0, _RPW, step=_NBUF)
        def _pipeline(i):
            for s in range(_NBUF):
                r = i + s
                wait(r, s)
                nxt = r + (_NBUF - 1)

                @pl.when(nxt < _RPW)
                def _():
                    fire(nxt, (s + _NBUF - 1) % _NBUF)

                _sum_rows(buf_v.at[s], r, out_v, corr, scale)

        pltpu.sync_copy(out_v, out_hbm.at[pl.ds(row_base, _RPW)])

    return sc_avg(idx2, embed)


_BM = 512


def _head_call(avg, W1, b1, W2, b2):
    W1a = W1[:, :_D]
    W1b = W1[:, _D:]
    w2p = jnp.zeros((128, _HID), jnp.float32).at[:_NCLASS].set(W2)
    b2p = jnp.zeros((1, 128), jnp.float32).at[0, :_NCLASS].set(b2)
    b1r = b1.reshape(1, _HID)

    def head(x1_ref, x2_ref, w1a_ref, w1b_ref, b1_ref, w2_ref, b2_ref, o_ref):
        h = lax.dot_general(x1_ref[...], w1a_ref[...], (((1,), (1,)), ((), ())),
                            preferred_element_type=jnp.float32)
        h = h + lax.dot_general(x2_ref[...], w1b_ref[...], (((1,), (1,)), ((), ())),
                                preferred_element_type=jnp.float32)
        h = jnp.maximum(h + b1_ref[...], 0.0)
        o = lax.dot_general(h, w2_ref[...], (((1,), (1,)), ((), ())),
                            preferred_element_type=jnp.float32)
        o_ref[...] = o + b2_ref[...]

    nblk = _B // _BM
    out = pl.pallas_call(
        head,
        grid=(nblk,),
        in_specs=[
            pl.BlockSpec((_BM, _D), lambda g: (g, 0)),
            pl.BlockSpec((_BM, _D), lambda g: (g + nblk, 0)),
            pl.BlockSpec((_HID, _D), lambda g: (0, 0)),
            pl.BlockSpec((_HID, _D), lambda g: (0, 0)),
            pl.BlockSpec((1, _HID), lambda g: (0, 0)),
            pl.BlockSpec((128, _HID), lambda g: (0, 0)),
            pl.BlockSpec((1, 128), lambda g: (0, 0)),
        ],
        out_specs=pl.BlockSpec((_BM, 128), lambda g: (g, 0)),
        out_shape=jax.ShapeDtypeStruct((_B, 128), jnp.float32),
    )(avg, avg, W1a, W1b, b1r, w2p, b2p)
    return out[:, :_NCLASS]


def kernel(arg1, arg2, embed, W1, b1, W2, b2):
    idx = jnp.concatenate([arg1.astype(jnp.int32), arg2.astype(jnp.int32)],
                          axis=0)
    idx = jnp.pad(idx, ((0, 0), (0, _LP - _L)))
    idx2 = idx.reshape(_ROWS * 2, _CHUNK)
    avg = _sc_avg_call(idx2, embed)
    return _head_call(avg, W1, b1, W2, b2)


# speedup vs baseline: 2.7088x; 2.7088x over previous
import functools

import jax
import jax.numpy as jnp
from jax import lax
from jax.experimental import pallas as pl
from jax.experimental.pallas import tpu as pltpu
from jax.experimental.pallas import tpu_sc as plsc

_B = 4096
_L = 200
_D = 64
_HID = 256
_NCLASS = 4

_NC, _NS = 2, 16
_NW = _NC * _NS

_C0, _C1 = 96, 104
_ROWS = 2 * _B
_RPW = _ROWS // _NW
_NBUF = 4


def _sum_rows(buf, r, out_v, scale):
    zeros = jnp.zeros((16,), jnp.float32)

    def body(j, acc):
        a = list(acc)
        for u in range(4):
            row = j * 4 + u
            for k in range(4):
                a[k] = a[k] + buf[row, pl.ds(k * 16, 16)]
        return tuple(a)

    acc = lax.fori_loop(0, _L // 4, body, (zeros,) * 4)
    for k in range(4):
        out_v[r, pl.ds(k * 16, 16)] = acc[k] * scale


def _sc_avg_call(arg1, arg2, embed):
    mesh = plsc.VectorSubcoreMesh(core_axis_name="c", subcore_axis_name="s",
                                  num_cores=_NC, num_subcores=_NS)

    @functools.partial(
        pl.kernel,
        out_type=jax.ShapeDtypeStruct((_ROWS, _D), jnp.float32),
        mesh=mesh,
        compiler_params=pltpu.CompilerParams(use_tc_tiling_on_sc=False),
        scratch_types=[
            pltpu.VMEM((_RPW, _L), jnp.int32),
            pltpu.VMEM((_NBUF, _L, _D), jnp.float32),
            pltpu.VMEM((_RPW, _D), jnp.float32),
            pltpu.SemaphoreType.DMA,
            pltpu.SemaphoreType.DMA,
            pltpu.SemaphoreType.DMA,
            pltpu.SemaphoreType.DMA,
        ],
    )
    def sc_avg(a1_hbm, a2_hbm, embed_hbm, out_hbm, idx_v, buf_v, out_v,
               s0, s1, s2, s3):
        sems = (s0, s1, s2, s3)
        wid = lax.axis_index("s") * _NC + lax.axis_index("c")
        half = _NW // 2
        row_base = wid * _RPW

        @pl.when(wid < half)
        def _():
            pltpu.sync_copy(a1_hbm.at[pl.ds(wid * _RPW, _RPW)], idx_v)

        @pl.when(wid >= half)
        def _():
            pltpu.sync_copy(a2_hbm.at[pl.ds((wid - half) * _RPW, _RPW)], idx_v)

        scale = jnp.float32(1.0 / _L)

        def fire(r, b):
            pltpu.async_copy(
                embed_hbm.at[idx_v.at[r, pl.ds(0, _C0)]],
                buf_v.at[b, pl.ds(0, _C0)],
                sems[b],
            )
            pltpu.async_copy(
                embed_hbm.at[idx_v.at[r, pl.ds(_C0, _C1)]],
                buf_v.at[b, pl.ds(_C0, _C1)],
                sems[b],
            )

        def wait(r, b):
            pltpu.make_async_copy(
                embed_hbm.at[idx_v.at[r, pl.ds(0, _C0)]],
                buf_v.at[b, pl.ds(0, _C0)],
                sems[b],
            ).wait()
            pltpu.make_async_copy(
                embed_hbm.at[idx_v.at[r, pl.ds(_C0, _C1)]],
                buf_v.at[b, pl.ds(_C0, _C1)],
                sems[b],
            ).wait()

        for b in range(_NBUF - 1):
            fire(b, b)

        @pl.loop(0, _RPW, step=_NBUF)
        def _pipeline(i):
            for s in range(_NBUF):
                r = i + s
                wait(r, s)
                nxt = r + (_NBUF - 1)

                @pl.when(nxt < _RPW)
                def _():
                    fire(nxt, (s + _NBUF - 1) % _NBUF)

                _sum_rows(buf_v.at[s], r, out_v, scale)

        pltpu.sync_copy(out_v, out_hbm.at[pl.ds(row_base, _RPW)])

    return sc_avg(arg1, arg2, embed)


_BM = 512


def _head_call(avg, W1, b1, W2, b2):
    W1a = W1[:, :_D]
    W1b = W1[:, _D:]
    w2p = jnp.zeros((128, _HID), jnp.float32).at[:_NCLASS].set(W2)
    b2p = jnp.zeros((1, 128), jnp.float32).at[0, :_NCLASS].set(b2)
    b1r = b1.reshape(1, _HID)

    def head(x1_ref, x2_ref, w1a_ref, w1b_ref, b1_ref, w2_ref, b2_ref, o_ref):
        h = lax.dot_general(x1_ref[...], w1a_ref[...], (((1,), (1,)), ((), ())),
                            preferred_element_type=jnp.float32)
        h = h + lax.dot_general(x2_ref[...], w1b_ref[...], (((1,), (1,)), ((), ())),
                                preferred_element_type=jnp.float32)
        h = jnp.maximum(h + b1_ref[...], 0.0)
        o = lax.dot_general(h, w2_ref[...], (((1,), (1,)), ((), ())),
                            preferred_element_type=jnp.float32)
        o_ref[...] = o + b2_ref[...]

    nblk = _B // _BM
    out = pl.pallas_call(
        head,
        grid=(nblk,),
        in_specs=[
            pl.BlockSpec((_BM, _D), lambda g: (g, 0)),
            pl.BlockSpec((_BM, _D), lambda g: (g + nblk, 0)),
            pl.BlockSpec((_HID, _D), lambda g: (0, 0)),
            pl.BlockSpec((_HID, _D), lambda g: (0, 0)),
            pl.BlockSpec((1, _HID), lambda g: (0, 0)),
            pl.BlockSpec((128, _HID), lambda g: (0, 0)),
            pl.BlockSpec((1, 128), lambda g: (0, 0)),
        ],
        out_specs=pl.BlockSpec((_BM, 128), lambda g: (g, 0)),
        out_shape=jax.ShapeDtypeStruct((_B, 128), jnp.float32),
    )(avg, avg, W1a, W1b, b1r, w2p, b2p)
    return out[:, :_NCLASS]


def kernel(arg1, arg2, embed, W1, b1, W2, b2):
    avg = _sc_avg_call(arg1.astype(jnp.int32), arg2.astype(jnp.int32), embed)
    return _head_call(avg, W1, b1, W2, b2)
